# baseline (device time: 42776 ns/iter reference)
import os

import jax
import jax.numpy as jnp
from jax import lax
from jax.experimental import pallas as pl
from jax.experimental.pallas import tpu as pltpu

N_DEV = 4
T = 1024
_ABLATE = os.environ.get("KERNEL_ABLATE", "")


def _piece_offsets(i):
    h_a = ((i == 1) | (i == 2)).astype(jnp.int32)
    w_a = i // 2
    h_b = i // 2
    w_b = i % 2
    a_half = h_a * 256
    a_oth = (1 - h_a) * 256
    a_q = a_half + w_a * 128
    a_sq = a_half + (1 - w_a) * 128
    b_half = 512 + h_b * 256
    b_oth = 512 + (1 - h_b) * 256
    b_q = b_half + w_b * 128
    b_sq = b_half + (1 - w_b) * 128
    return a_half, a_oth, a_q, a_sq, b_half, b_oth, b_q, b_sq


def _body(packed_ref, n_ref, e_ref, out_ref,
          staging, work, scratch, send_sems, recv_sems, g1_sem, g2_sem):
    i = lax.axis_index("i")
    pa = i ^ 1
    pb = 3 - i
    n_send = n_ref[0]
    n_mine = n_ref[1]

    a_half, a_oth, a_q, a_sq, b_half, b_oth, b_q, b_sq = _piece_offsets(i)

    staging[...] = jnp.zeros(staging.shape, jnp.float32)

    def issue_send_region(k, carry):
        v = packed_ref[k]
        tok = (v >> 14) & 1023
        row = v & 16383
        pltpu.make_async_copy(e_ref.at[row], staging.at[tok], g1_sem).start()
        return carry

    def issue_kept(k, carry):
        v = packed_ref[k]
        tok = (v >> 14) & 1023
        row = v & 16383
        pltpu.make_async_copy(e_ref.at[row], staging.at[tok], g2_sem).start()
        return carry

    if _ABLATE != "nogather":
        lax.fori_loop(0, n_send, issue_send_region, 0)
        lax.fori_loop(n_send, n_mine, issue_kept, 0)

    barrier_sem = pltpu.get_barrier_semaphore()
    for nbr in (pa, pb):
        pl.semaphore_signal(
            barrier_sem, inc=1,
            device_id=(nbr,), device_id_type=pl.DeviceIdType.MESH,
        )
    pl.semaphore_wait(barrier_sem, 2)

    def exch(k, partner, src, dst):
        return pltpu.make_async_remote_copy(
            src_ref=src, dst_ref=dst,
            send_sem=send_sems.at[k], recv_sem=recv_sems.at[k],
            device_id=(partner,), device_id_type=pl.DeviceIdType.MESH,
        )

    def drain(sem):
        def one(k, carry):
            pltpu.make_async_copy(e_ref.at[0], staging.at[0], sem).wait()
            return carry
        return one

    bf16 = jnp.bfloat16

    def to_bf16(off, n=128):
        work[pl.ds(off, n), :] = staging[pl.ds(off, n), :].astype(bf16)

    def to_f32(off, n=128):
        out_ref[pl.ds(off, n), :] = work[pl.ds(off, n), :].astype(jnp.float32)

    def add(off, scr_off, n=128):
        work[pl.ds(off, n), :] += scratch[pl.ds(scr_off, n), :]

    if _ABLATE != "nogather":
        lax.fori_loop(0, n_send, drain(g1_sem), 0)
    to_bf16(a_oth, 256)
    to_bf16(b_oth, 256)
    if _ABLATE == "nocomm":
        if _ABLATE != "nogather":
            lax.fori_loop(n_send, n_mine, drain(g2_sem), 0)
        out_ref[...] = staging[...]
        return
    s1 = [
        exch(0, pa, work.at[pl.ds(a_oth, 128)], scratch.at[pl.ds(0, 128)]),
        exch(1, pa, work.at[pl.ds(a_oth + 128, 128)], scratch.at[pl.ds(128, 128)]),
        exch(2, pb, work.at[pl.ds(b_oth, 128)], scratch.at[pl.ds(256, 128)]),
        exch(3, pb, work.at[pl.ds(b_oth + 128, 128)], scratch.at[pl.ds(384, 128)]),
    ]
    for r in (s1[0], s1[2], s1[1], s1[3]):
        r.start()
    if _ABLATE != "nogather":
        lax.fori_loop(n_send, n_mine, drain(g2_sem), 0)
    to_bf16(a_half, 256)
    to_bf16(b_half, 256)

    s2 = [
        exch(4, pb, work.at[pl.ds(a_half, 128)], scratch.at[pl.ds(512, 128)]),
        exch(5, pb, work.at[pl.ds(a_half + 128, 128)], scratch.at[pl.ds(640, 128)]),
        exch(6, pa, work.at[pl.ds(b_half, 128)], scratch.at[pl.ds(768, 128)]),
        exch(7, pa, work.at[pl.ds(b_half + 128, 128)], scratch.at[pl.ds(896, 128)]),
    ]
    s4 = [
        exch(8, pa, work.at[pl.ds(a_half, 128)], work.at[pl.ds(a_half, 128)]),
        exch(9, pa, work.at[pl.ds(a_half + 128, 128)], work.at[pl.ds(a_half + 128, 128)]),
        exch(10, pb, work.at[pl.ds(b_half, 128)], work.at[pl.ds(b_half, 128)]),
        exch(11, pb, work.at[pl.ds(b_half + 128, 128)], work.at[pl.ds(b_half + 128, 128)]),
    ]
    s1[0].wait()
    add(a_half, 0)
    s1[1].wait()
    add(a_half + 128, 128)
    s2[0].start()
    s2[1].start()
    s1[2].wait()
    add(b_half, 256)
    s1[3].wait()
    add(b_half + 128, 384)
    s2[2].start()
    s2[3].start()

    s2[0].wait()
    add(a_half, 512)
    s2[1].wait()
    add(a_half + 128, 640)
    s4[0].start()
    s4[1].start()
    to_f32(a_half, 256)
    s2[2].wait()
    add(b_half, 768)
    s2[3].wait()
    add(b_half + 128, 896)
    s4[2].start()
    s4[3].start()
    to_f32(b_half, 256)

    s4[0].wait()
    to_f32(a_oth)
    s4[1].wait()
    to_f32(a_oth + 128)
    s4[2].wait()
    to_f32(b_oth)
    s4[3].wait()
    to_f32(b_oth + 128)


def kernel(ids, E):
    v_loc, d = E.shape

    my_pos = lax.axis_index("i")
    local_ids = ids - my_pos * v_loc
    in_range = (local_ids >= 0) & (local_ids < v_loc)

    _, a_oth, _, _, _, b_oth, _, _ = _piece_offsets(my_pos)
    tok = jnp.arange(T, dtype=jnp.int32)
    in_send = ((tok >= a_oth) & (tok < a_oth + 256)) | (
        (tok >= b_oth) & (tok < b_oth + 256)
    )

    prio = jnp.where(in_range, jnp.where(in_send, 0, 1), 2)
    key = prio * 1024 + tok
    packed = jnp.sort((key << 14) | jnp.where(in_range, local_ids, 0))
    n_send = jnp.sum((in_range & in_send).astype(jnp.int32))
    n_mine = jnp.sum(in_range.astype(jnp.int32))
    counts = jnp.stack([n_send, n_mine])

    grid_spec = pltpu.PrefetchScalarGridSpec(
        num_scalar_prefetch=2,
        grid=(),
        in_specs=[
            pl.BlockSpec(memory_space=pltpu.MemorySpace.HBM),
        ],
        out_specs=pl.BlockSpec(memory_space=pltpu.VMEM),
        scratch_shapes=[
            pltpu.VMEM((T, d), jnp.float32),
            pltpu.VMEM((T, d), jnp.bfloat16),
            pltpu.VMEM((1024, d), jnp.bfloat16),
            pltpu.SemaphoreType.DMA((12,)),
            pltpu.SemaphoreType.DMA((12,)),
            pltpu.SemaphoreType.DMA,
            pltpu.SemaphoreType.DMA,
        ],
    )
    return pl.pallas_call(
        _body,
        grid_spec=grid_spec,
        out_shape=jax.ShapeDtypeStruct((T, d), jnp.float32),
        compiler_params=pltpu.CompilerParams(collective_id=0),
    )(packed, counts, E)


# device time: 42038 ns/iter; 1.0176x vs baseline; 1.0176x over previous
import os

import jax
import jax.numpy as jnp
from jax import lax
from jax.experimental import pallas as pl
from jax.experimental.pallas import tpu as pltpu

N_DEV = 4
T = 1024
_ABLATE = os.environ.get("KERNEL_ABLATE", "")


def _piece_offsets(i):
    h_a = ((i == 1) | (i == 2)).astype(jnp.int32)
    w_a = i // 2
    h_b = i // 2
    w_b = i % 2
    a_half = h_a * 256
    a_oth = (1 - h_a) * 256
    a_q = a_half + w_a * 128
    a_sq = a_half + (1 - w_a) * 128
    b_half = 512 + h_b * 256
    b_oth = 512 + (1 - h_b) * 256
    b_q = b_half + w_b * 128
    b_sq = b_half + (1 - w_b) * 128
    return a_half, a_oth, a_q, a_sq, b_half, b_oth, b_q, b_sq


def _body(packed_ref, n_ref, e_ref, out_ref,
          staging, work, scratch, send_sems, recv_sems, g1_sem, g2_sem):
    i = lax.axis_index("i")
    pa = i ^ 1
    pb = 3 - i
    n_send = n_ref[0]
    n_mine = n_ref[1]

    a_half, a_oth, a_q, a_sq, b_half, b_oth, b_q, b_sq = _piece_offsets(i)

    staging[...] = jnp.zeros(staging.shape, jnp.float32)

    def issue_send_region(k, carry):
        v = packed_ref[k]
        tok = (v >> 14) & 1023
        row = v & 16383
        pltpu.make_async_copy(e_ref.at[row], staging.at[tok], g1_sem).start()
        return carry

    def issue_kept(k, carry):
        v = packed_ref[k]
        tok = (v >> 14) & 1023
        row = v & 16383
        pltpu.make_async_copy(e_ref.at[row], staging.at[tok], g2_sem).start()
        return carry

    if _ABLATE != "nogather":
        lax.fori_loop(0, n_send, issue_send_region, 0)
        lax.fori_loop(n_send, n_mine, issue_kept, 0)

    barrier_sem = pltpu.get_barrier_semaphore()
    for nbr in (pa, pb):
        pl.semaphore_signal(
            barrier_sem, inc=1,
            device_id=(nbr,), device_id_type=pl.DeviceIdType.MESH,
        )
    pl.semaphore_wait(barrier_sem, 2)

    def exch(k, partner, src, dst):
        return pltpu.make_async_remote_copy(
            src_ref=src, dst_ref=dst,
            send_sem=send_sems.at[k], recv_sem=recv_sems.at[k],
            device_id=(partner,), device_id_type=pl.DeviceIdType.MESH,
        )

    def drain(sem):
        def one(k, carry):
            pltpu.make_async_copy(e_ref.at[0], staging.at[0], sem).wait()
            return carry
        return one

    bf16 = jnp.bfloat16

    def to_bf16(off, n=128):
        work[pl.ds(off, n), :] = staging[pl.ds(off, n), :].astype(bf16)

    def to_f32(off, n=128):
        out_ref[pl.ds(off, n), :] = work[pl.ds(off, n), :].astype(jnp.float32)

    def add(off, scr_off, n=128):
        work[pl.ds(off, n), :] += scratch[pl.ds(scr_off, n), :]

    if _ABLATE != "nogather":
        lax.fori_loop(0, n_send, drain(g1_sem), 0)
    to_bf16(a_oth, 256)
    to_bf16(b_oth, 256)
    if _ABLATE == "nocomm":
        if _ABLATE != "nogather":
            lax.fori_loop(n_send, n_mine, drain(g2_sem), 0)
        out_ref[...] = staging[...]
        return
    s1 = [
        exch(0, pa, work.at[pl.ds(a_oth, 128)], scratch.at[pl.ds(0, 128)]),
        exch(1, pa, work.at[pl.ds(a_oth + 128, 128)], scratch.at[pl.ds(128, 128)]),
        exch(2, pb, work.at[pl.ds(b_oth, 128)], scratch.at[pl.ds(256, 128)]),
        exch(3, pb, work.at[pl.ds(b_oth + 128, 128)], scratch.at[pl.ds(384, 128)]),
    ]
    for r in (s1[0], s1[2], s1[1], s1[3]):
        r.start()
    if _ABLATE != "nogather":
        lax.fori_loop(n_send, n_mine, drain(g2_sem), 0)
    to_bf16(a_half, 256)
    to_bf16(b_half, 256)

    s2 = [
        exch(4, pb, work.at[pl.ds(a_half, 128)], scratch.at[pl.ds(512, 128)]),
        exch(5, pb, work.at[pl.ds(a_half + 128, 128)], scratch.at[pl.ds(640, 128)]),
        exch(6, pa, work.at[pl.ds(b_half, 128)], scratch.at[pl.ds(768, 128)]),
        exch(7, pa, work.at[pl.ds(b_half + 128, 128)], scratch.at[pl.ds(896, 128)]),
    ]
    s4 = [
        exch(8, pa, work.at[pl.ds(a_half, 128)], work.at[pl.ds(a_half, 128)]),
        exch(9, pa, work.at[pl.ds(a_half + 128, 128)], work.at[pl.ds(a_half + 128, 128)]),
        exch(10, pb, work.at[pl.ds(b_half, 128)], work.at[pl.ds(b_half, 128)]),
        exch(11, pb, work.at[pl.ds(b_half + 128, 128)], work.at[pl.ds(b_half + 128, 128)]),
    ]
    s1[0].wait()
    add(a_half, 0)
    s1[1].wait()
    add(a_half + 128, 128)
    s2[0].start()
    s2[1].start()
    s1[2].wait()
    add(b_half, 256)
    s1[3].wait()
    add(b_half + 128, 384)
    s2[2].start()
    s2[3].start()

    s2[0].wait()
    add(a_half, 512)
    s2[1].wait()
    add(a_half + 128, 640)
    s4[0].start()
    s4[1].start()
    to_f32(a_half, 256)
    s2[2].wait()
    add(b_half, 768)
    s2[3].wait()
    add(b_half + 128, 896)
    s4[2].start()
    s4[3].start()
    to_f32(b_half, 256)

    s4[0].wait()
    to_f32(a_oth)
    s4[1].wait()
    to_f32(a_oth + 128)
    s4[2].wait()
    to_f32(b_oth)
    s4[3].wait()
    to_f32(b_oth + 128)


def kernel(ids, E):
    v_loc, d = E.shape

    my_pos = lax.axis_index("i")
    local_ids = ids - my_pos * v_loc
    in_range = (local_ids >= 0) & (local_ids < v_loc)

    _, a_oth, _, _, _, b_oth, _, _ = _piece_offsets(my_pos)
    tok = jnp.arange(T, dtype=jnp.int32)
    in_send = ((tok >= a_oth) & (tok < a_oth + 256)) | (
        (tok >= b_oth) & (tok < b_oth + 256)
    )

    prio = jnp.where(in_range, jnp.where(in_send, 0, 1), 2)
    key = prio * 1024 + tok
    packed = jnp.sort(
        (key << 14) | jnp.where(in_range, local_ids, 0), stable=False
    )
    n_send = jnp.sum((in_range & in_send).astype(jnp.int32))
    n_mine = jnp.sum(in_range.astype(jnp.int32))
    counts = jnp.stack([n_send, n_mine])

    grid_spec = pltpu.PrefetchScalarGridSpec(
        num_scalar_prefetch=2,
        grid=(),
        in_specs=[
            pl.BlockSpec(memory_space=pltpu.MemorySpace.HBM),
        ],
        out_specs=pl.BlockSpec(memory_space=pltpu.VMEM),
        scratch_shapes=[
            pltpu.VMEM((T, d), jnp.float32),
            pltpu.VMEM((T, d), jnp.bfloat16),
            pltpu.VMEM((1024, d), jnp.bfloat16),
            pltpu.SemaphoreType.DMA((12,)),
            pltpu.SemaphoreType.DMA((12,)),
            pltpu.SemaphoreType.DMA,
            pltpu.SemaphoreType.DMA,
        ],
    )
    return pl.pallas_call(
        _body,
        grid_spec=grid_spec,
        out_shape=jax.ShapeDtypeStruct((T, d), jnp.float32),
        compiler_params=pltpu.CompilerParams(collective_id=0),
    )(packed, counts, E)


# device time: 38976 ns/iter; 1.0975x vs baseline; 1.0786x over previous
import os

import jax
import jax.numpy as jnp
from jax import lax
from jax.experimental import pallas as pl
from jax.experimental.pallas import tpu as pltpu

N_DEV = 4
T = 1024
_ABLATE = os.environ.get("KERNEL_ABLATE", "")


def _piece_offsets(i):
    h_a = ((i == 1) | (i == 2)).astype(jnp.int32)
    w_a = i // 2
    h_b = i // 2
    w_b = i % 2
    a_half = h_a * 256
    a_oth = (1 - h_a) * 256
    a_q = a_half + w_a * 128
    a_sq = a_half + (1 - w_a) * 128
    b_half = 512 + h_b * 256
    b_oth = 512 + (1 - h_b) * 256
    b_q = b_half + w_b * 128
    b_sq = b_half + (1 - w_b) * 128
    return a_half, a_oth, a_q, a_sq, b_half, b_oth, b_q, b_sq


def _body(packed_ref, n_ref, e_ref, out_ref,
          staging, work, scratch, send_sems, recv_sems, g1_sem, g2_sem):
    i = lax.axis_index("i")
    pa = i ^ 1
    pb = 3 - i
    n_send = n_ref[0]
    n_mine = n_ref[1]

    a_half, a_oth, a_q, a_sq, b_half, b_oth, b_q, b_sq = _piece_offsets(i)

    zero = jnp.zeros((256, staging.shape[1]), jnp.float32)
    staging[pl.ds(a_oth, 256), :] = zero
    staging[pl.ds(b_oth, 256), :] = zero

    def issue_send_region(k, carry):
        v = packed_ref[k]
        tok = (v >> 14) & 1023
        row = v & 16383
        pltpu.make_async_copy(e_ref.at[row], staging.at[tok], g1_sem).start()
        return carry

    def issue_kept(k, carry):
        v = packed_ref[k]
        tok = (v >> 14) & 1023
        row = v & 16383
        pltpu.make_async_copy(e_ref.at[row], staging.at[tok], g2_sem).start()
        return carry

    if _ABLATE != "nogather":
        lax.fori_loop(0, n_send, issue_send_region, 0)

    barrier_sem = pltpu.get_barrier_semaphore()
    for nbr in (pa, pb):
        pl.semaphore_signal(
            barrier_sem, inc=1,
            device_id=(nbr,), device_id_type=pl.DeviceIdType.MESH,
        )
    pl.semaphore_wait(barrier_sem, 2)

    def exch(k, partner, src, dst):
        return pltpu.make_async_remote_copy(
            src_ref=src, dst_ref=dst,
            send_sem=send_sems.at[k], recv_sem=recv_sems.at[k],
            device_id=(partner,), device_id_type=pl.DeviceIdType.MESH,
        )

    def drain(sem):
        def one(k, carry):
            pltpu.make_async_copy(e_ref.at[0], staging.at[0], sem).wait()
            return carry
        return one

    bf16 = jnp.bfloat16

    def to_bf16(off, n=128):
        work[pl.ds(off, n), :] = staging[pl.ds(off, n), :].astype(bf16)

    def to_f32(off, n=128):
        out_ref[pl.ds(off, n), :] = work[pl.ds(off, n), :].astype(jnp.float32)

    def add(off, scr_off, n=128):
        work[pl.ds(off, n), :] += scratch[pl.ds(scr_off, n), :]

    if _ABLATE != "nogather":
        lax.fori_loop(0, n_send, drain(g1_sem), 0)
    to_bf16(a_oth, 256)
    to_bf16(b_oth, 256)
    if _ABLATE == "nocomm":
        staging[pl.ds(a_half, 256), :] = zero
        staging[pl.ds(b_half, 256), :] = zero
        if _ABLATE != "nogather":
            lax.fori_loop(n_send, n_mine, issue_kept, 0)
            lax.fori_loop(n_send, n_mine, drain(g2_sem), 0)
        out_ref[...] = staging[...]
        return
    s1 = [
        exch(0, pa, work.at[pl.ds(a_oth, 128)], scratch.at[pl.ds(0, 128)]),
        exch(1, pa, work.at[pl.ds(a_oth + 128, 128)], scratch.at[pl.ds(128, 128)]),
        exch(2, pb, work.at[pl.ds(b_oth, 128)], scratch.at[pl.ds(256, 128)]),
        exch(3, pb, work.at[pl.ds(b_oth + 128, 128)], scratch.at[pl.ds(384, 128)]),
    ]
    for r in (s1[0], s1[2], s1[1], s1[3]):
        r.start()
    staging[pl.ds(a_half, 256), :] = zero
    staging[pl.ds(b_half, 256), :] = zero
    if _ABLATE != "nogather":
        lax.fori_loop(n_send, n_mine, issue_kept, 0)
        lax.fori_loop(n_send, n_mine, drain(g2_sem), 0)
    to_bf16(a_half, 256)
    to_bf16(b_half, 256)

    s2 = [
        exch(4, pb, work.at[pl.ds(a_half, 128)], scratch.at[pl.ds(512, 128)]),
        exch(5, pb, work.at[pl.ds(a_half + 128, 128)], scratch.at[pl.ds(640, 128)]),
        exch(6, pa, work.at[pl.ds(b_half, 128)], scratch.at[pl.ds(768, 128)]),
        exch(7, pa, work.at[pl.ds(b_half + 128, 128)], scratch.at[pl.ds(896, 128)]),
    ]
    s4 = [
        exch(8, pa, work.at[pl.ds(a_half, 128)], work.at[pl.ds(a_half, 128)]),
        exch(9, pa, work.at[pl.ds(a_half + 128, 128)], work.at[pl.ds(a_half + 128, 128)]),
        exch(10, pb, work.at[pl.ds(b_half, 128)], work.at[pl.ds(b_half, 128)]),
        exch(11, pb, work.at[pl.ds(b_half + 128, 128)], work.at[pl.ds(b_half + 128, 128)]),
    ]
    s1[0].wait()
    add(a_half, 0)
    s1[1].wait()
    add(a_half + 128, 128)
    s2[0].start()
    s2[1].start()
    s1[2].wait()
    add(b_half, 256)
    s1[3].wait()
    add(b_half + 128, 384)
    s2[2].start()
    s2[3].start()

    s2[0].wait()
    add(a_half, 512)
    s2[1].wait()
    add(a_half + 128, 640)
    s4[0].start()
    s4[1].start()
    to_f32(a_half, 256)
    s2[2].wait()
    add(b_half, 768)
    s2[3].wait()
    add(b_half + 128, 896)
    s4[2].start()
    s4[3].start()
    to_f32(b_half, 256)

    s4[0].wait()
    to_f32(a_oth)
    s4[1].wait()
    to_f32(a_oth + 128)
    s4[2].wait()
    to_f32(b_oth)
    s4[3].wait()
    to_f32(b_oth + 128)


def kernel(ids, E):
    v_loc, d = E.shape

    my_pos = lax.axis_index("i")
    local_ids = ids - my_pos * v_loc
    in_range = (local_ids >= 0) & (local_ids < v_loc)

    _, a_oth, _, _, _, b_oth, _, _ = _piece_offsets(my_pos)
    tok = jnp.arange(T, dtype=jnp.int32)
    in_send = ((tok >= a_oth) & (tok < a_oth + 256)) | (
        (tok >= b_oth) & (tok < b_oth + 256)
    )

    prio = jnp.where(in_range, jnp.where(in_send, 0, 1), 2)
    key = prio * 1024 + tok
    packed = jnp.sort(
        (key << 14) | jnp.where(in_range, local_ids, 0), stable=False
    )
    n_send = jnp.sum((in_range & in_send).astype(jnp.int32))
    n_mine = jnp.sum(in_range.astype(jnp.int32))
    counts = jnp.stack([n_send, n_mine])

    grid_spec = pltpu.PrefetchScalarGridSpec(
        num_scalar_prefetch=2,
        grid=(),
        in_specs=[
            pl.BlockSpec(memory_space=pltpu.MemorySpace.HBM),
        ],
        out_specs=pl.BlockSpec(memory_space=pltpu.VMEM),
        scratch_shapes=[
            pltpu.VMEM((T, d), jnp.float32),
            pltpu.VMEM((T, d), jnp.bfloat16),
            pltpu.VMEM((1024, d), jnp.bfloat16),
            pltpu.SemaphoreType.DMA((12,)),
            pltpu.SemaphoreType.DMA((12,)),
            pltpu.SemaphoreType.DMA,
            pltpu.SemaphoreType.DMA,
        ],
    )
    return pl.pallas_call(
        _body,
        grid_spec=grid_spec,
        out_shape=jax.ShapeDtypeStruct((T, d), jnp.float32),
        compiler_params=pltpu.CompilerParams(collective_id=0),
    )(packed, counts, E)
